# Initial kernel scaffold; baseline (speedup 1.0000x reference)
import jax
import jax.numpy as jnp
from jax.experimental import pallas as pl


def kernel(edge_index, user_emb, item_emb):
    raise NotImplementedError("write your pallas kernel here")


# speedup vs baseline: 1.8571x; 1.8571x over previous
import functools

import jax
import jax.numpy as jnp
from jax.experimental import pallas as pl

N_NODES = 10000
N_PAD = 10240
DIM = 256
BM = 256


def _mm_body(x0_ref, y_ref, a_ref, out_ref, *, scale):
    acc = jnp.dot(a_ref[...], y_ref[...], preferred_element_type=jnp.float32)
    out_ref[...] = (x0_ref[...] + acc) * scale


def _propagate(a, x0, y, scale):
    grid = (N_PAD // BM,)
    return pl.pallas_call(
        functools.partial(_mm_body, scale=scale),
        grid=grid,
        in_specs=[
            pl.BlockSpec((BM, DIM), lambda i: (i, 0)),
            pl.BlockSpec((N_PAD, DIM), lambda i: (0, 0)),
            pl.BlockSpec((BM, N_PAD), lambda i: (i, 0)),
        ],
        out_specs=pl.BlockSpec((BM, DIM), lambda i: (i, 0)),
        out_shape=jax.ShapeDtypeStruct((N_PAD, DIM), jnp.float32),
    )(x0, y, a)


def kernel(edge_index, user_emb, item_emb):
    n_users = user_emb.shape[0]
    row = edge_index[0].astype(jnp.int32)
    col = edge_index[1].astype(jnp.int32)

    deg = jnp.zeros((N_PAD,), jnp.float32).at[col].add(1.0)
    dis = jnp.where(deg > 0, jax.lax.rsqrt(deg), 0.0)
    norm = dis[row] * dis[col]

    a = jnp.zeros((N_PAD, N_PAD), jnp.float32).at[col, row].add(norm)

    x0 = jnp.concatenate(
        [user_emb, item_emb,
         jnp.zeros((N_PAD - N_NODES, DIM), jnp.float32)], axis=0)

    y = _propagate(a, x0, x0, 1.0)
    y = _propagate(a, x0, y, 1.0)
    final = _propagate(a, x0, y, 0.25)

    return (final[:n_users], final[n_users:N_NODES])
